# baseline (speedup 1.0000x reference)
import jax
import jax.numpy as jnp
from jax.experimental import pallas as pl
from jax.experimental.pallas import tpu as pltpu

IN_FEATURES = 1024
HIDDEN = 128
OUT = 1
BATCH = 16384
BATCH_TILE = 2048
NSPLIT = 2
COLS = IN_FEATURES // NSPLIT

_K1 = max(1, int((1.0 - 0.6) * (HIDDEN * IN_FEATURES)))
_K2 = max(1, int((1.0 - 0.7) * (HIDDEN * HIDDEN)))
_K3 = max(1, int((1.0 - 0.6) * (OUT * HIDDEN)))


def _kth_bits(u, k):
    def body(i, t):
        cand = t | (jnp.int32(1) << (jnp.int32(30) - i))
        cnt = jnp.sum((u >= cand).astype(jnp.int32))
        return jnp.where(cnt >= k, cand, t)

    return jax.lax.fori_loop(0, 31, body, jnp.int32(0))


def _mask_body(w1_ref, w2_ref, w3_ref, m1_ref, m2_ref, m3_ref):
    for w_ref, m_ref, k, transpose in (
        (w1_ref, m1_ref, _K1, True),
        (w2_ref, m2_ref, _K2, True),
        (w3_ref, m3_ref, _K3, False),
    ):
        w = w_ref[...]
        u = jax.lax.bitcast_convert_type(jnp.abs(w), jnp.int32)
        t = _kth_bits(u, k)
        m = jnp.where(u >= t, w, 0.0)
        if transpose:
            m = m.T
        m_ref[...] = m.astype(m_ref.dtype)


def _mlp_body(*refs):
    x_refs = refs[:NSPLIT]
    w1_ref, b1_ref, w2_ref, b2_ref, w3_ref, b3_ref, o_ref = refs[NSPLIT:]
    dn = (((1,), (0,)), ((), ()))
    h = jax.lax.dot_general(
        x_refs[0][...].astype(jnp.bfloat16), w1_ref[0:COLS, :], dn,
        preferred_element_type=jnp.float32)
    for s in range(1, NSPLIT):
        h = h + jax.lax.dot_general(
            x_refs[s][...].astype(jnp.bfloat16),
            w1_ref[s * COLS:(s + 1) * COLS, :], dn,
            preferred_element_type=jnp.float32)
    h = jnp.maximum(h + b1_ref[...], 0.0).astype(jnp.bfloat16)
    h = jax.lax.dot_general(h, w2_ref[...], dn,
                            preferred_element_type=jnp.float32)
    h = jnp.maximum(h + b2_ref[...], 0.0)
    o = jnp.sum(h * w3_ref[...], axis=1, keepdims=True)
    o_ref[...] = o + b3_ref[0, 0]


def kernel(X, W1, b1, W2, b2, W3, b3):
    masks = pl.pallas_call(
        _mask_body,
        out_shape=(
            jax.ShapeDtypeStruct((IN_FEATURES, HIDDEN), jnp.bfloat16),
            jax.ShapeDtypeStruct((HIDDEN, HIDDEN), jnp.bfloat16),
            jax.ShapeDtypeStruct(W3.shape, jnp.float32),
        ),
    )(W1, W2, W3)
    W1m, W2m, W3m = masks

    b1r = b1.reshape(1, HIDDEN)
    b2r = b2.reshape(1, HIDDEN)
    b3r = b3.reshape(1, OUT)

    def x_spec(s):
        return pl.BlockSpec((BATCH_TILE, COLS), lambda i, s=s: (i, s))

    grid = (BATCH // BATCH_TILE,)
    out = pl.pallas_call(
        _mlp_body,
        grid=grid,
        in_specs=[x_spec(s) for s in range(NSPLIT)] + [
            pl.BlockSpec((IN_FEATURES, HIDDEN), lambda i: (0, 0)),
            pl.BlockSpec((1, HIDDEN), lambda i: (0, 0)),
            pl.BlockSpec((HIDDEN, HIDDEN), lambda i: (0, 0)),
            pl.BlockSpec((1, HIDDEN), lambda i: (0, 0)),
            pl.BlockSpec((OUT, HIDDEN), lambda i: (0, 0)),
            pl.BlockSpec(memory_space=pltpu.SMEM),
        ],
        out_specs=pl.BlockSpec((BATCH_TILE, OUT), lambda i: (i, 0)),
        out_shape=jax.ShapeDtypeStruct((BATCH, OUT), jnp.float32),
        compiler_params=pltpu.CompilerParams(
            dimension_semantics=("parallel",)),
    )(*([X] * NSPLIT), W1m, b1r, W2m, b2r, W3m, b3r)
    return out


# speedup vs baseline: 1.0415x; 1.0415x over previous
import jax
import jax.numpy as jnp
from jax.experimental import pallas as pl
from jax.experimental.pallas import tpu as pltpu

IN_FEATURES = 1024
HIDDEN = 128
OUT = 1
BATCH = 16384
BATCH_TILE = 2048

_K1 = max(1, int((1.0 - 0.6) * (HIDDEN * IN_FEATURES)))
_K2 = max(1, int((1.0 - 0.7) * (HIDDEN * HIDDEN)))
_K3 = max(1, int((1.0 - 0.6) * (OUT * HIDDEN)))


def _kth_bits(u, k):
    def body(i, t):
        cand = t | (jnp.int32(1) << (jnp.int32(30) - i))
        cnt = jnp.sum((u >= cand).astype(jnp.int32))
        return jnp.where(cnt >= k, cand, t)

    return jax.lax.fori_loop(0, 31, body, jnp.int32(0))


def _fused_body(x_ref, w1_ref, b1_ref, w2_ref, b2_ref, w3_ref, b3_ref,
                o_ref, w1s, w2s, w3s):
    i = pl.program_id(0)

    @pl.when(i == 0)
    def _compute_masks():
        for w_ref, m_ref, k, transpose in (
            (w1_ref, w1s, _K1, True),
            (w2_ref, w2s, _K2, True),
            (w3_ref, w3s, _K3, False),
        ):
            w = w_ref[...]
            u = jax.lax.bitcast_convert_type(jnp.abs(w), jnp.int32)
            t = _kth_bits(u, k)
            m = jnp.where(u >= t, w, 0.0)
            if transpose:
                m = m.T
            m_ref[...] = m.astype(m_ref.dtype)

    dn = (((1,), (0,)), ((), ()))
    h = jax.lax.dot_general(x_ref[...].astype(jnp.bfloat16), w1s[...], dn,
                            preferred_element_type=jnp.float32)
    h = jnp.maximum(h + b1_ref[...], 0.0).astype(jnp.bfloat16)
    h = jax.lax.dot_general(h, w2s[...], dn,
                            preferred_element_type=jnp.float32)
    h = jnp.maximum(h + b2_ref[...], 0.0)
    o = jnp.sum(h * w3s[...], axis=1, keepdims=True)
    o_ref[...] = o + b3_ref[0, 0]


def kernel(X, W1, b1, W2, b2, W3, b3):
    b1r = b1.reshape(1, HIDDEN)
    b2r = b2.reshape(1, HIDDEN)
    b3r = b3.reshape(1, OUT)

    grid = (BATCH // BATCH_TILE,)
    out = pl.pallas_call(
        _fused_body,
        grid=grid,
        in_specs=[
            pl.BlockSpec((BATCH_TILE, IN_FEATURES), lambda i: (i, 0)),
            pl.BlockSpec((HIDDEN, IN_FEATURES), lambda i: (0, 0)),
            pl.BlockSpec((1, HIDDEN), lambda i: (0, 0)),
            pl.BlockSpec((HIDDEN, HIDDEN), lambda i: (0, 0)),
            pl.BlockSpec((1, HIDDEN), lambda i: (0, 0)),
            pl.BlockSpec((OUT, HIDDEN), lambda i: (0, 0)),
            pl.BlockSpec(memory_space=pltpu.SMEM),
        ],
        out_specs=pl.BlockSpec((BATCH_TILE, OUT), lambda i: (i, 0)),
        out_shape=jax.ShapeDtypeStruct((BATCH, OUT), jnp.float32),
        scratch_shapes=[
            pltpu.VMEM((IN_FEATURES, HIDDEN), jnp.bfloat16),
            pltpu.VMEM((HIDDEN, HIDDEN), jnp.bfloat16),
            pltpu.VMEM((OUT, HIDDEN), jnp.float32),
        ],
        compiler_params=pltpu.CompilerParams(
            dimension_semantics=("arbitrary",)),
    )(X, W1, b1r, W2, b2r, W3, b3r)
    return out


# speedup vs baseline: 1.2323x; 1.1833x over previous
import jax
import jax.numpy as jnp
from jax.experimental import pallas as pl
from jax.experimental.pallas import tpu as pltpu

IN_FEATURES = 1024
HIDDEN = 128
OUT = 1
BATCH = 16384
BATCH_TILE = 2048

_K1 = max(1, int((1.0 - 0.6) * (HIDDEN * IN_FEATURES)))
_K2 = max(1, int((1.0 - 0.7) * (HIDDEN * HIDDEN)))
_K3 = max(1, int((1.0 - 0.6) * (OUT * HIDDEN)))


def _kth_bits(u, k):
    def body(i, t):
        cand = t | (jnp.int32(1) << (jnp.int32(30) - i))
        cnt = jnp.sum((u >= cand).astype(jnp.int32))
        return jnp.where(cnt >= k, cand, t)

    return jax.lax.fori_loop(0, 31, body, jnp.int32(0))


def _fused_body(x_ref, w1_ref, b1_ref, w2_ref, b2_ref, w3_ref, b3_ref,
                o_ref, w1s, w2s, w3s):
    i = pl.program_id(0)

    @pl.when(i == 0)
    def _compute_masks():
        w1 = w1_ref[...]
        w2 = w2_ref[...]
        w3 = w3_ref[...]
        u1 = jax.lax.bitcast_convert_type(jnp.abs(w1), jnp.int32)
        u2 = jax.lax.bitcast_convert_type(jnp.abs(w2), jnp.int32)
        u3 = jax.lax.bitcast_convert_type(jnp.abs(w3), jnp.int32)

        def body(idx, ts):
            t1, t2, t3 = ts
            bit = jnp.int32(1) << (jnp.int32(30) - idx)
            c1 = t1 | bit
            c2 = t2 | bit
            c3 = t3 | bit
            n1 = jnp.sum((u1 >= c1).astype(jnp.int32))
            n2 = jnp.sum((u2 >= c2).astype(jnp.int32))
            n3 = jnp.sum((u3 >= c3).astype(jnp.int32))
            return (jnp.where(n1 >= _K1, c1, t1),
                    jnp.where(n2 >= _K2, c2, t2),
                    jnp.where(n3 >= _K3, c3, t3))

        z = jnp.int32(0)
        t1, t2, t3 = jax.lax.fori_loop(0, 31, body, (z, z, z))
        w1s[...] = jnp.where(u1 >= t1, w1, 0.0).T.astype(w1s.dtype)
        w2s[...] = jnp.where(u2 >= t2, w2, 0.0).T.astype(w2s.dtype)
        w3s[...] = jnp.where(u3 >= t3, w3, 0.0).astype(w3s.dtype)

    dn = (((1,), (0,)), ((), ()))
    h = jax.lax.dot_general(x_ref[...].astype(jnp.bfloat16), w1s[...], dn,
                            preferred_element_type=jnp.float32)
    h = jnp.maximum(h + b1_ref[...], 0.0).astype(jnp.bfloat16)
    h = jax.lax.dot_general(h, w2s[...], dn,
                            preferred_element_type=jnp.float32)
    h = jnp.maximum(h + b2_ref[...], 0.0)
    o = jnp.sum(h * w3s[...], axis=1, keepdims=True)
    o_ref[...] = o + b3_ref[0, 0]


def kernel(X, W1, b1, W2, b2, W3, b3):
    b1r = b1.reshape(1, HIDDEN)
    b2r = b2.reshape(1, HIDDEN)
    b3r = b3.reshape(1, OUT)

    grid = (BATCH // BATCH_TILE,)
    out = pl.pallas_call(
        _fused_body,
        grid=grid,
        in_specs=[
            pl.BlockSpec((BATCH_TILE, IN_FEATURES), lambda i: (i, 0)),
            pl.BlockSpec((HIDDEN, IN_FEATURES), lambda i: (0, 0)),
            pl.BlockSpec((1, HIDDEN), lambda i: (0, 0)),
            pl.BlockSpec((HIDDEN, HIDDEN), lambda i: (0, 0)),
            pl.BlockSpec((1, HIDDEN), lambda i: (0, 0)),
            pl.BlockSpec((OUT, HIDDEN), lambda i: (0, 0)),
            pl.BlockSpec(memory_space=pltpu.SMEM),
        ],
        out_specs=pl.BlockSpec((BATCH_TILE, OUT), lambda i: (i, 0)),
        out_shape=jax.ShapeDtypeStruct((BATCH, OUT), jnp.float32),
        scratch_shapes=[
            pltpu.VMEM((IN_FEATURES, HIDDEN), jnp.bfloat16),
            pltpu.VMEM((HIDDEN, HIDDEN), jnp.bfloat16),
            pltpu.VMEM((OUT, HIDDEN), jnp.float32),
        ],
        compiler_params=pltpu.CompilerParams(
            dimension_semantics=("arbitrary",)),
    )(X, W1, b1r, W2, b2r, W3, b3r)
    return out


# speedup vs baseline: 1.2730x; 1.0330x over previous
import jax
import jax.numpy as jnp
from jax.experimental import pallas as pl
from jax.experimental.pallas import tpu as pltpu

IN_FEATURES = 1024
HIDDEN = 128
OUT = 1
BATCH = 16384
BATCH_TILE = 2048
NTILES = BATCH // BATCH_TILE
NBUF = 3

_K1 = max(1, int((1.0 - 0.6) * (HIDDEN * IN_FEATURES)))
_K2 = max(1, int((1.0 - 0.7) * (HIDDEN * HIDDEN)))
_K3 = max(1, int((1.0 - 0.6) * (OUT * HIDDEN)))


def _fused_body(x_hbm, w1_ref, b1_ref, w2_ref, b2_ref, w3_ref, b3_ref,
                o_ref, xb, sems):

    def tile_copy(t, slot):
        return pltpu.make_async_copy(
            x_hbm.at[pl.ds(t * BATCH_TILE, BATCH_TILE), :],
            xb.at[slot], sems.at[slot])

    for t in range(NBUF):
        tile_copy(t, t).start()

    w1 = w1_ref[...]
    w2 = w2_ref[...]
    w3 = w3_ref[...]
    u1 = jax.lax.bitcast_convert_type(jnp.abs(w1), jnp.int32)
    u2 = jax.lax.bitcast_convert_type(jnp.abs(w2), jnp.int32)
    u3 = jax.lax.bitcast_convert_type(jnp.abs(w3), jnp.int32)

    def body(idx, ts):
        t1, t2, t3 = ts
        bit = jnp.int32(1) << (jnp.int32(30) - idx)
        c1 = t1 | bit
        c2 = t2 | bit
        c3 = t3 | bit
        n1 = jnp.sum((u1 >= c1).astype(jnp.int32))
        n2 = jnp.sum((u2 >= c2).astype(jnp.int32))
        n3 = jnp.sum((u3 >= c3).astype(jnp.int32))
        return (jnp.where(n1 >= _K1, c1, t1),
                jnp.where(n2 >= _K2, c2, t2),
                jnp.where(n3 >= _K3, c3, t3))

    z = jnp.int32(0)
    t1, t2, t3 = jax.lax.fori_loop(0, 31, body, (z, z, z))
    w1m = jnp.where(u1 >= t1, w1, 0.0).T.astype(jnp.bfloat16)
    w2m = jnp.where(u2 >= t2, w2, 0.0).T.astype(jnp.bfloat16)
    w3m = jnp.where(u3 >= t3, w3, 0.0)

    dn = (((1,), (0,)), ((), ()))
    for t in range(NTILES):
        slot = t % NBUF
        tile_copy(t, slot).wait()
        h = jax.lax.dot_general(xb[slot].astype(jnp.bfloat16), w1m, dn,
                                preferred_element_type=jnp.float32)
        h = jnp.maximum(h + b1_ref[...], 0.0).astype(jnp.bfloat16)
        h = jax.lax.dot_general(h, w2m, dn,
                                preferred_element_type=jnp.float32)
        h = jnp.maximum(h + b2_ref[...], 0.0)
        o = jnp.sum(h * w3m, axis=1, keepdims=True)
        o_ref[pl.ds(t * BATCH_TILE, BATCH_TILE), :] = o + b3_ref[0, 0]
        if t + NBUF < NTILES:
            tile_copy(t + NBUF, slot).start()


def kernel(X, W1, b1, W2, b2, W3, b3):
    b1r = b1.reshape(1, HIDDEN)
    b2r = b2.reshape(1, HIDDEN)
    b3r = b3.reshape(1, OUT)

    out = pl.pallas_call(
        _fused_body,
        in_specs=[
            pl.BlockSpec(memory_space=pl.ANY),
            pl.BlockSpec(memory_space=pltpu.MemorySpace.VMEM),
            pl.BlockSpec(memory_space=pltpu.MemorySpace.VMEM),
            pl.BlockSpec(memory_space=pltpu.MemorySpace.VMEM),
            pl.BlockSpec(memory_space=pltpu.MemorySpace.VMEM),
            pl.BlockSpec(memory_space=pltpu.MemorySpace.VMEM),
            pl.BlockSpec(memory_space=pltpu.MemorySpace.SMEM),
        ],
        out_specs=pl.BlockSpec(memory_space=pltpu.MemorySpace.VMEM),
        out_shape=jax.ShapeDtypeStruct((BATCH, OUT), jnp.float32),
        scratch_shapes=[
            pltpu.VMEM((NBUF, BATCH_TILE, IN_FEATURES), jnp.float32),
            pltpu.SemaphoreType.DMA((NBUF,)),
        ],
    )(X, W1, b1r, W2, b2r, W3, b3r)
    return out


# speedup vs baseline: 1.2736x; 1.0005x over previous
import jax
import jax.numpy as jnp
from jax.experimental import pallas as pl
from jax.experimental.pallas import tpu as pltpu

IN_FEATURES = 1024
HIDDEN = 128
OUT = 1
BATCH = 16384
BATCH_TILE = 2048
NTILES = BATCH // BATCH_TILE
NBUF = 4

_K1 = max(1, int((1.0 - 0.6) * (HIDDEN * IN_FEATURES)))
_K2 = max(1, int((1.0 - 0.7) * (HIDDEN * HIDDEN)))
_K3 = max(1, int((1.0 - 0.6) * (OUT * HIDDEN)))


def _fused_body(x_hbm, w1_ref, b1_ref, w2_ref, b2_ref, w3_ref, b3_ref,
                o_ref, xb, sems):

    def tile_copy(t, slot):
        return pltpu.make_async_copy(
            x_hbm.at[pl.ds(t * BATCH_TILE, BATCH_TILE), :],
            xb.at[slot], sems.at[slot])

    for t in range(NBUF):
        tile_copy(t, t).start()

    w1 = w1_ref[...]
    w2 = w2_ref[...]
    w3 = w3_ref[...]
    u1 = jax.lax.bitcast_convert_type(jnp.abs(w1), jnp.int32)
    u2 = jax.lax.bitcast_convert_type(jnp.abs(w2), jnp.int32)
    u3 = jax.lax.bitcast_convert_type(jnp.abs(w3), jnp.int32)

    def body(idx, ts):
        t1, t2, t3 = ts
        bit = jnp.int32(1) << (jnp.int32(30) - idx)
        c1 = t1 | bit
        c2 = t2 | bit
        c3 = t3 | bit
        n1 = jnp.sum((u1 >= c1).astype(jnp.int32))
        n2 = jnp.sum((u2 >= c2).astype(jnp.int32))
        n3 = jnp.sum((u3 >= c3).astype(jnp.int32))
        return (jnp.where(n1 >= _K1, c1, t1),
                jnp.where(n2 >= _K2, c2, t2),
                jnp.where(n3 >= _K3, c3, t3))

    z = jnp.int32(0)
    t1, t2, t3 = jax.lax.fori_loop(0, 31, body, (z, z, z))
    w1m = jnp.where(u1 >= t1, w1, 0.0).T.astype(jnp.bfloat16)
    w2m = jnp.where(u2 >= t2, w2, 0.0).T.astype(jnp.bfloat16)
    w3m = jnp.where(u3 >= t3, w3, 0.0)

    dn = (((1,), (0,)), ((), ()))
    for t in range(NTILES):
        slot = t % NBUF
        tile_copy(t, slot).wait()
        h = jax.lax.dot_general(xb[slot].astype(jnp.bfloat16), w1m, dn,
                                preferred_element_type=jnp.float32)
        h = jnp.maximum(h + b1_ref[...], 0.0).astype(jnp.bfloat16)
        h = jax.lax.dot_general(h, w2m, dn,
                                preferred_element_type=jnp.float32)
        h = jnp.maximum(h + b2_ref[...], 0.0)
        o = jnp.sum(h * w3m, axis=1, keepdims=True)
        o_ref[pl.ds(t * BATCH_TILE, BATCH_TILE), :] = o + b3_ref[0, 0]
        if t + NBUF < NTILES:
            tile_copy(t + NBUF, slot).start()


def kernel(X, W1, b1, W2, b2, W3, b3):
    b1r = b1.reshape(1, HIDDEN)
    b2r = b2.reshape(1, HIDDEN)
    b3r = b3.reshape(1, OUT)

    out = pl.pallas_call(
        _fused_body,
        in_specs=[
            pl.BlockSpec(memory_space=pl.ANY),
            pl.BlockSpec(memory_space=pltpu.MemorySpace.VMEM),
            pl.BlockSpec(memory_space=pltpu.MemorySpace.VMEM),
            pl.BlockSpec(memory_space=pltpu.MemorySpace.VMEM),
            pl.BlockSpec(memory_space=pltpu.MemorySpace.VMEM),
            pl.BlockSpec(memory_space=pltpu.MemorySpace.VMEM),
            pl.BlockSpec(memory_space=pltpu.MemorySpace.SMEM),
        ],
        out_specs=pl.BlockSpec(memory_space=pltpu.MemorySpace.VMEM),
        out_shape=jax.ShapeDtypeStruct((BATCH, OUT), jnp.float32),
        scratch_shapes=[
            pltpu.VMEM((NBUF, BATCH_TILE, IN_FEATURES), jnp.float32),
            pltpu.SemaphoreType.DMA((NBUF,)),
        ],
    )(X, W1, b1r, W2, b2r, W3, b3r)
    return out
